# baseline (speedup 1.0000x reference)
import functools

import jax
import jax.numpy as jnp
from jax import lax
from jax.experimental import pallas as pl
from jax.experimental.pallas import tpu as pltpu
from jax.experimental.pallas import tpu_sc as plsc

NC = 2
NS = 16
NW = NC * NS

U = 256
D = 64


@functools.lru_cache(maxsize=None)
def _build(B, S, V2):
    NB = B // S
    stripe = NB // NW
    units_per_j = stripe // U
    NU = S * units_per_j
    mesh = plsc.VectorSubcoreMesh(
        core_axis_name="c", subcore_axis_name="s", num_cores=NC, num_subcores=NS
    )

    @functools.partial(
        pl.kernel,
        mesh=mesh,
        out_type=jax.ShapeDtypeStruct((S, D, NB), jnp.float32),
        scratch_types=[
            pltpu.VMEM((U,), jnp.int32),
            pltpu.VMEM((U,), jnp.int32),
            pltpu.VMEM((U,), jnp.int32),
            pltpu.VMEM((U,), jnp.int32),
            pltpu.VMEM((U, 2 * D), jnp.float32),
            pltpu.VMEM((U, 2 * D), jnp.float32),
            pltpu.VMEM((D, U), jnp.float32),
            pltpu.VMEM((D, U), jnp.float32),
            pltpu.SemaphoreType.DMA,
            pltpu.SemaphoreType.DMA,
            pltpu.SemaphoreType.DMA,
            pltpu.SemaphoreType.DMA,
            pltpu.SemaphoreType.DMA,
        ],
        compiler_params=pltpu.CompilerParams(needs_layout_passes=False),
    )
    def emb(pair_hbm, half_hbm, tab_hbm, out_hbm,
            pair0, pair1, half0, half1, buf0, buf1, bt0, bt1,
            isem, g0, g1, w0, w1):
        wid = lax.axis_index("s") * NC + lax.axis_index("c")
        ubase = wid * NU
        col_base = wid * stripe
        pair_v = (pair0, pair1)
        half_v = (half0, half1)
        buf_v = (buf0, buf1)
        bt_v = (bt0, bt1)
        gsem = (g0, g1)
        wsem = (w0, w1)
        iota16 = lax.iota(jnp.int32, 16)

        def fire_idx(n, p):
            off = (ubase + n) * U
            pltpu.async_copy(pair_hbm.at[pl.ds(off, U)], pair_v[p], isem)
            pltpu.async_copy(half_hbm.at[pl.ds(off, U)], half_v[p], isem)

        def wait_idx(p):
            pltpu.make_async_copy(
                pair_hbm.at[pl.ds(0, U)], pair_v[p], isem).wait()
            pltpu.make_async_copy(
                half_hbm.at[pl.ds(0, U)], half_v[p], isem).wait()

        def fire_gathers(p):
            for c in range(U // 128):
                pltpu.async_copy(
                    tab_hbm.at[pair_v[p].at[pl.ds(c * 128, 128)]],
                    buf_v[p].at[pl.ds(c * 128, 128)],
                    gsem[p],
                )

        def wait_gathers(p):
            pltpu.make_async_copy(
                tab_hbm.at[pl.ds(0, U)], buf_v[p], gsem[p]).wait()

        def transpose(p):
            buf = buf_v[p]
            bt = bt_v[p]
            halfs = [half_v[p][pl.ds(mb * 16, 16)] for mb in range(U // 16)]
            rows = [iota16 + (mb * 16) for mb in range(U // 16)]

            def body(k, carry):
                for mb in range(U // 16):
                    vec = plsc.load_gather(buf, [rows[mb], halfs[mb] + k])
                    bt[k, pl.ds(mb * 16, 16)] = vec
                return carry

            lax.fori_loop(0, D, body, 0)

        def fire_writes(n, p):
            j = n // units_per_j
            h = n % units_per_j
            col0 = col_base + h * U
            bt = bt_v[p]
            for kb in range(D // 8):
                pltpu.async_copy(
                    bt.at[pl.ds(kb * 8, 8)],
                    out_hbm.at[j, pl.ds(kb * 8, 8), pl.ds(col0, U)],
                    wsem[p],
                )

        def wait_writes(p):
            pltpu.make_async_copy(
                out_hbm.at[0, :, pl.ds(0, U)], bt_v[p], wsem[p]).wait()

        fire_idx(0, 0)
        wait_idx(0)
        fire_gathers(0)
        fire_idx(1, 1)

        wait_gathers(0)
        wait_idx(1)
        fire_gathers(1)
        transpose(0)
        fire_idx(2, 0)
        fire_writes(0, 0)
        wait_gathers(1)
        wait_idx(0)
        fire_gathers(0)
        transpose(1)
        fire_idx(3, 1)
        fire_writes(1, 1)

        def step(t, carry):
            for p in range(2):
                n = 2 * t + p
                wait_writes(p)
                wait_gathers(p)
                wait_idx(1 - p)
                fire_gathers(1 - p)
                transpose(p)
                fire_idx(n + 2, p)
                fire_writes(n, p)
            return carry

        lax.fori_loop(1, NU // 2 - 1, step, 0)

        n = NU - 2
        wait_writes(0)
        wait_gathers(0)
        wait_idx(1)
        fire_gathers(1)
        transpose(0)
        fire_writes(n, 0)
        wait_writes(1)
        wait_gathers(1)
        transpose(1)
        fire_writes(n + 1, 1)
        wait_writes(0)
        wait_writes(1)

    return emb


def kernel(token_ids, weight):
    NB, S = token_ids.shape
    V, Dm = weight.shape
    B = NB * S
    stripe = NB // NW
    ids = token_ids.astype(jnp.int32)
    perm = ids.T.reshape(S, NW, stripe // U, U).transpose(1, 0, 2, 3).reshape(-1)
    pair = perm >> 1
    half = (perm & 1) << 6
    tab2 = weight.reshape(V // 2, 2 * Dm)
    emb = _build(B, S, V // 2)
    out = emb(pair, half, tab2)
    return out.transpose(2, 0, 1)


# speedup vs baseline: 1.7338x; 1.7338x over previous
import functools

import jax
import jax.numpy as jnp
from jax import lax
from jax.experimental import pallas as pl
from jax.experimental.pallas import tpu as pltpu
from jax.experimental.pallas import tpu_sc as plsc

NC = 2
NS = 16
NW = NC * NS

U = 256
D = 64


@functools.lru_cache(maxsize=None)
def _build(B, S, V2):
    NB = B // S
    stripe = NB // NW
    units_per_j = stripe // U
    NU = S * units_per_j
    mesh = plsc.VectorSubcoreMesh(
        core_axis_name="c", subcore_axis_name="s", num_cores=NC, num_subcores=NS
    )

    @functools.partial(
        pl.kernel,
        mesh=mesh,
        out_type=jax.ShapeDtypeStruct((S, D, NB), jnp.float32),
        scratch_types=[
            pltpu.VMEM((U,), jnp.int32),
            pltpu.VMEM((U,), jnp.int32),
            pltpu.VMEM((U,), jnp.int32),
            pltpu.VMEM((U,), jnp.int32),
            pltpu.VMEM((U, 2 * D), jnp.float32),
            pltpu.VMEM((U, 2 * D), jnp.float32),
            pltpu.VMEM((D, U), jnp.float32),
            pltpu.VMEM((D, U), jnp.float32),
            pltpu.SemaphoreType.DMA,
            pltpu.SemaphoreType.DMA,
            pltpu.SemaphoreType.DMA,
            pltpu.SemaphoreType.DMA,
            pltpu.SemaphoreType.DMA,
        ],
        compiler_params=pltpu.CompilerParams(needs_layout_passes=False),
    )
    def emb(pair_hbm, half_hbm, tab_hbm, out_hbm,
            pair0, pair1, half0, half1, buf0, buf1, bt0, bt1,
            isem, g0, g1, w0, w1):
        wid = lax.axis_index("s") * NC + lax.axis_index("c")
        ubase = wid * NU
        col_base = wid * stripe
        pair_v = (pair0, pair1)
        half_v = (half0, half1)
        buf_v = (buf0, buf1)
        bt_v = (bt0, bt1)
        gsem = (g0, g1)
        wsem = (w0, w1)
        iota16 = lax.iota(jnp.int32, 16)

        def fire_idx(n, p):
            off = (ubase + n) * U
            pltpu.async_copy(pair_hbm.at[pl.ds(off, U)], pair_v[p], isem)
            pltpu.async_copy(half_hbm.at[pl.ds(off, U)], half_v[p], isem)

        def wait_idx(p):
            pltpu.make_async_copy(
                pair_hbm.at[pl.ds(0, U)], pair_v[p], isem).wait()
            pltpu.make_async_copy(
                half_hbm.at[pl.ds(0, U)], half_v[p], isem).wait()

        def fire_gathers(p):
            for c in range(U // 128):
                pltpu.async_copy(
                    tab_hbm.at[pair_v[p].at[pl.ds(c * 128, 128)]],
                    buf_v[p].at[pl.ds(c * 128, 128)],
                    gsem[p],
                )

        def wait_gathers(p):
            pltpu.make_async_copy(
                tab_hbm.at[pl.ds(0, U)], buf_v[p], gsem[p]).wait()

        def transpose(p):
            buf = buf_v[p]
            bt = bt_v[p]
            halfs = [half_v[p][pl.ds(mb * 16, 16)] for mb in range(U // 16)]
            rows = [iota16 + (mb * 16) for mb in range(U // 16)]

            def body(c, carry):
                rot = (iota16 + c) & 15
                for kb in range(D // 16):
                    rowk = rot + (kb * 16)
                    for mb in range(U // 16):
                        vec = plsc.load_gather(buf, [rows[mb], halfs[mb] + rowk])
                        plsc.store_scatter(bt, [rowk, rows[mb]], vec)
                return carry

            lax.fori_loop(0, 16, body, 0)

        def fire_writes(n, p):
            j = n // units_per_j
            h = n % units_per_j
            col0 = col_base + h * U
            bt = bt_v[p]
            for kb in range(D // 8):
                pltpu.async_copy(
                    bt.at[pl.ds(kb * 8, 8)],
                    out_hbm.at[j, pl.ds(kb * 8, 8), pl.ds(col0, U)],
                    wsem[p],
                )

        def wait_writes(p):
            pltpu.make_async_copy(
                out_hbm.at[0, :, pl.ds(0, U)], bt_v[p], wsem[p]).wait()

        fire_idx(0, 0)
        wait_idx(0)
        fire_gathers(0)
        fire_idx(1, 1)

        wait_gathers(0)
        wait_idx(1)
        fire_gathers(1)
        transpose(0)
        fire_idx(2, 0)
        fire_writes(0, 0)
        wait_gathers(1)
        wait_idx(0)
        fire_gathers(0)
        transpose(1)
        fire_idx(3, 1)
        fire_writes(1, 1)

        def step(t, carry):
            for p in range(2):
                n = 2 * t + p
                wait_writes(p)
                wait_gathers(p)
                wait_idx(1 - p)
                fire_gathers(1 - p)
                transpose(p)
                fire_idx(n + 2, p)
                fire_writes(n, p)
            return carry

        lax.fori_loop(1, NU // 2 - 1, step, 0)

        n = NU - 2
        wait_writes(0)
        wait_gathers(0)
        wait_idx(1)
        fire_gathers(1)
        transpose(0)
        fire_writes(n, 0)
        wait_writes(1)
        wait_gathers(1)
        transpose(1)
        fire_writes(n + 1, 1)
        wait_writes(0)
        wait_writes(1)

    return emb


def kernel(token_ids, weight):
    NB, S = token_ids.shape
    V, Dm = weight.shape
    B = NB * S
    stripe = NB // NW
    ids = token_ids.astype(jnp.int32)
    perm = ids.T.reshape(S, NW, stripe // U, U).transpose(1, 0, 2, 3).reshape(-1)
    pair = perm >> 1
    half = (perm & 1) << 6
    tab2 = weight.reshape(V // 2, 2 * Dm)
    emb = _build(B, S, V // 2)
    out = emb(pair, half, tab2)
    return out.transpose(2, 0, 1)


# speedup vs baseline: 1.9220x; 1.1085x over previous
import functools

import jax
import jax.numpy as jnp
from jax import lax
from jax.experimental import pallas as pl
from jax.experimental.pallas import tpu as pltpu
from jax.experimental.pallas import tpu_sc as plsc

NC = 2
NS = 16
NW = NC * NS

U = 256
D = 64


@functools.lru_cache(maxsize=None)
def _build_relayout(V, Dm):
    n_full = V // 128
    n_tail = V - n_full * 128
    slots = 246
    mesh = plsc.VectorSubcoreMesh(
        core_axis_name="c", subcore_axis_name="s", num_cores=NC, num_subcores=NS
    )

    @functools.partial(
        pl.kernel,
        mesh=mesh,
        out_type=jax.ShapeDtypeStruct((V // 2, 2 * Dm), jnp.float32),
        scratch_types=[
            pltpu.VMEM((Dm, 128), jnp.float32),
            pltpu.VMEM((Dm, 128), jnp.float32),
            pltpu.VMEM((64, 2 * Dm), jnp.float32),
            pltpu.VMEM((64, 2 * Dm), jnp.float32),
            pltpu.VMEM((n_tail // 2, 2 * Dm), jnp.float32),
            pltpu.SemaphoreType.DMA,
            pltpu.SemaphoreType.DMA,
            pltpu.SemaphoreType.DMA,
            pltpu.SemaphoreType.DMA,
        ],
        compiler_params=pltpu.CompilerParams(needs_layout_passes=False),
    )
    def relayout(wt_hbm, tail_hbm, tab_hbm, in0, in1, ot0, ot1, tbounce,
                 r0, r1, w0, w1):
        wid = lax.axis_index("s") * NC + lax.axis_index("c")
        in_v = (in0, in1)
        ot_v = (ot0, ot1)
        rsem = (r0, r1)
        wsem = (w0, w1)
        iota16 = lax.iota(jnp.int32, 16)

        def tb_of(n):
            raw = n * NW + wid
            return jnp.where(raw < n_full, raw, 0)

        def fire_read(n, p):
            tb = tb_of(n)
            for kb in range(Dm // 8):
                pltpu.async_copy(
                    wt_hbm.at[pl.ds(kb * 8, 8), pl.ds(tb * 128, 128)],
                    in_v[p].at[pl.ds(kb * 8, 8)],
                    rsem[p],
                )

        def wait_read(p):
            pltpu.make_async_copy(
                wt_hbm.at[pl.ds(0, Dm), pl.ds(0, 128)], in_v[p], rsem[p]).wait()

        cols = [iota16 + (ic * 16) for ic in range(128 // 16)]
        prow = [c >> 1 for c in cols]
        phalf = [(c & 1) << 6 for c in cols]

        def transpose(p):
            inb = in_v[p]
            ot = ot_v[p]

            def body(c, carry):
                rot = (iota16 + c) & 15
                for kc in range(Dm // 16):
                    rowk = rot + (kc * 16)
                    for ic in range(128 // 16):
                        vec = plsc.load_gather(inb, [rowk, cols[ic]])
                        plsc.store_scatter(
                            ot, [prow[ic], phalf[ic] + rowk], vec)
                return carry

            lax.fori_loop(0, 16, body, 0)

        def fire_write(n, p):
            pltpu.async_copy(
                ot_v[p],
                tab_hbm.at[pl.ds(tb_of(n) * 64, 64)],
                wsem[p],
            )

        def wait_write(p):
            pltpu.make_async_copy(
                tab_hbm.at[pl.ds(0, 64)], ot_v[p], wsem[p]).wait()

        fire_read(0, 0)
        fire_read(1, 1)
        wait_read(0)
        transpose(0)
        fire_read(2, 0)
        fire_write(0, 0)
        wait_read(1)
        transpose(1)
        fire_read(3, 1)
        fire_write(1, 1)

        def step(t, carry):
            for p in range(2):
                n = 2 * t + p
                wait_write(p)
                wait_read(p)
                transpose(p)
                fire_read(n + 2, p)
                fire_write(n, p)
            return carry

        lax.fori_loop(1, slots // 2 - 1, step, 0)

        n = slots - 2
        wait_write(0)
        wait_read(0)
        transpose(0)
        fire_write(n, 0)
        wait_write(1)
        wait_read(1)
        transpose(1)
        fire_write(n + 1, 1)
        wait_write(0)
        wait_write(1)

        @pl.when(wid == NW - 1)
        def _tail():
            pltpu.sync_copy(tail_hbm, tbounce)
            pltpu.sync_copy(
                tbounce, tab_hbm.at[pl.ds(n_full * 64, n_tail // 2)])

    return relayout


@functools.lru_cache(maxsize=None)
def _build(B, S, V2):
    NB = B // S
    stripe = NB // NW
    units_per_j = stripe // U
    NU = S * units_per_j
    mesh = plsc.VectorSubcoreMesh(
        core_axis_name="c", subcore_axis_name="s", num_cores=NC, num_subcores=NS
    )

    @functools.partial(
        pl.kernel,
        mesh=mesh,
        out_type=jax.ShapeDtypeStruct((S, D, NB), jnp.float32),
        scratch_types=[
            pltpu.VMEM((U,), jnp.int32),
            pltpu.VMEM((U,), jnp.int32),
            pltpu.VMEM((U,), jnp.int32),
            pltpu.VMEM((U,), jnp.int32),
            pltpu.VMEM((U, 2 * D), jnp.float32),
            pltpu.VMEM((U, 2 * D), jnp.float32),
            pltpu.VMEM((D, U), jnp.float32),
            pltpu.VMEM((D, U), jnp.float32),
            pltpu.SemaphoreType.DMA,
            pltpu.SemaphoreType.DMA,
            pltpu.SemaphoreType.DMA,
            pltpu.SemaphoreType.DMA,
            pltpu.SemaphoreType.DMA,
        ],
        compiler_params=pltpu.CompilerParams(needs_layout_passes=False),
    )
    def emb(pair_hbm, half_hbm, tab_hbm, out_hbm,
            pair0, pair1, half0, half1, buf0, buf1, bt0, bt1,
            isem, g0, g1, w0, w1):
        wid = lax.axis_index("s") * NC + lax.axis_index("c")
        ubase = wid * NU
        col_base = wid * stripe
        pair_v = (pair0, pair1)
        half_v = (half0, half1)
        buf_v = (buf0, buf1)
        bt_v = (bt0, bt1)
        gsem = (g0, g1)
        wsem = (w0, w1)
        iota16 = lax.iota(jnp.int32, 16)

        def fire_idx(n, p):
            off = (ubase + n) * U
            pltpu.async_copy(pair_hbm.at[pl.ds(off, U)], pair_v[p], isem)
            pltpu.async_copy(half_hbm.at[pl.ds(off, U)], half_v[p], isem)

        def wait_idx(p):
            pltpu.make_async_copy(
                pair_hbm.at[pl.ds(0, U)], pair_v[p], isem).wait()
            pltpu.make_async_copy(
                half_hbm.at[pl.ds(0, U)], half_v[p], isem).wait()

        def fire_gathers(p):
            for c in range(U // 128):
                pltpu.async_copy(
                    tab_hbm.at[pair_v[p].at[pl.ds(c * 128, 128)]],
                    buf_v[p].at[pl.ds(c * 128, 128)],
                    gsem[p],
                )

        def wait_gathers(p):
            pltpu.make_async_copy(
                tab_hbm.at[pl.ds(0, U)], buf_v[p], gsem[p]).wait()

        def transpose(p):
            buf = buf_v[p]
            bt = bt_v[p]
            halfs = [half_v[p][pl.ds(mb * 16, 16)] for mb in range(U // 16)]
            rows = [iota16 + (mb * 16) for mb in range(U // 16)]

            def body(c, carry):
                rot = (iota16 + c) & 15
                for kb in range(D // 16):
                    rowk = rot + (kb * 16)
                    for mb in range(U // 16):
                        vec = plsc.load_gather(buf, [rows[mb], halfs[mb] + rowk])
                        plsc.store_scatter(bt, [rowk, rows[mb]], vec)
                return carry

            lax.fori_loop(0, 16, body, 0)

        def fire_writes(n, p):
            j = n // units_per_j
            h = n % units_per_j
            col0 = col_base + h * U
            bt = bt_v[p]
            for kb in range(D // 8):
                pltpu.async_copy(
                    bt.at[pl.ds(kb * 8, 8)],
                    out_hbm.at[j, pl.ds(kb * 8, 8), pl.ds(col0, U)],
                    wsem[p],
                )

        def wait_writes(p):
            pltpu.make_async_copy(
                out_hbm.at[0, :, pl.ds(0, U)], bt_v[p], wsem[p]).wait()

        fire_idx(0, 0)
        wait_idx(0)
        fire_gathers(0)
        fire_idx(1, 1)

        wait_gathers(0)
        wait_idx(1)
        fire_gathers(1)
        transpose(0)
        fire_idx(2, 0)
        fire_writes(0, 0)
        wait_gathers(1)
        wait_idx(0)
        fire_gathers(0)
        transpose(1)
        fire_idx(3, 1)
        fire_writes(1, 1)

        def step(t, carry):
            for p in range(2):
                n = 2 * t + p
                wait_writes(p)
                wait_gathers(p)
                wait_idx(1 - p)
                fire_gathers(1 - p)
                transpose(p)
                fire_idx(n + 2, p)
                fire_writes(n, p)
            return carry

        lax.fori_loop(1, NU // 2 - 1, step, 0)

        n = NU - 2
        wait_writes(0)
        wait_gathers(0)
        wait_idx(1)
        fire_gathers(1)
        transpose(0)
        fire_writes(n, 0)
        wait_writes(1)
        wait_gathers(1)
        transpose(1)
        fire_writes(n + 1, 1)
        wait_writes(0)
        wait_writes(1)

    return emb


def kernel(token_ids, weight):
    NB, S = token_ids.shape
    V, Dm = weight.shape
    B = NB * S
    stripe = NB // NW
    ids = token_ids.astype(jnp.int32)
    perm = ids.T.reshape(S, NW, stripe // U, U).transpose(1, 0, 2, 3).reshape(-1)
    pair = perm >> 1
    half = (perm & 1) << 6
    n_full = (V // 128) * 128
    tail = weight[n_full:].reshape((V - n_full) // 2, 2 * Dm)
    tab2 = _build_relayout(V, Dm)(weight.T, tail)
    emb = _build(B, S, V // 2)
    out = emb(pair, half, tab2)
    return out.transpose(2, 0, 1)


# speedup vs baseline: 2.9356x; 1.5274x over previous
import functools

import jax
import jax.numpy as jnp
from jax import lax
from jax.experimental import pallas as pl
from jax.experimental.pallas import tpu as pltpu
from jax.experimental.pallas import tpu_sc as plsc

NC = 2
NS = 16
NW = NC * NS

U = 256
D = 64


@functools.lru_cache(maxsize=None)
def _build_relayout(V, Dm):
    n_full = V // 128
    n_tail = V - n_full * 128
    slots = 246
    mesh = plsc.VectorSubcoreMesh(
        core_axis_name="c", subcore_axis_name="s", num_cores=NC, num_subcores=NS
    )

    @functools.partial(
        pl.kernel,
        mesh=mesh,
        out_type=jax.ShapeDtypeStruct((V // 2, 2 * Dm), jnp.float32),
        scratch_types=[
            pltpu.VMEM((Dm, 128), jnp.float32),
            pltpu.VMEM((Dm, 128), jnp.float32),
            pltpu.VMEM((64, 2 * Dm), jnp.float32),
            pltpu.VMEM((64, 2 * Dm), jnp.float32),
            pltpu.VMEM((n_tail // 2, 2 * Dm), jnp.float32),
            pltpu.SemaphoreType.DMA,
            pltpu.SemaphoreType.DMA,
            pltpu.SemaphoreType.DMA,
            pltpu.SemaphoreType.DMA,
        ],
        compiler_params=pltpu.CompilerParams(needs_layout_passes=False),
    )
    def relayout(wt_hbm, tail_hbm, tab_hbm, in0, in1, ot0, ot1, tbounce,
                 r0, r1, w0, w1):
        wid = lax.axis_index("s") * NC + lax.axis_index("c")
        in_v = (in0, in1)
        ot_v = (ot0, ot1)
        rsem = (r0, r1)
        wsem = (w0, w1)
        iota16 = lax.iota(jnp.int32, 16)

        def tb_of(n):
            raw = n * NW + wid
            return jnp.where(raw < n_full, raw, 0)

        def fire_read(n, p):
            tb = tb_of(n)
            for kb in range(Dm // 8):
                pltpu.async_copy(
                    wt_hbm.at[pl.ds(kb * 8, 8), pl.ds(tb * 128, 128)],
                    in_v[p].at[pl.ds(kb * 8, 8)],
                    rsem[p],
                )

        def wait_read(p):
            pltpu.make_async_copy(
                wt_hbm.at[pl.ds(0, Dm), pl.ds(0, 128)], in_v[p], rsem[p]).wait()

        cols = [iota16 + (ic * 16) for ic in range(128 // 16)]
        prow = [c >> 1 for c in cols]
        phalf = [(c & 1) << 6 for c in cols]

        def transpose(p):
            inb = in_v[p]
            ot = ot_v[p]

            @plsc.parallel_loop(0, 16)
            def body(c):
                rot = (iota16 + c) & 15
                for kc in range(Dm // 16):
                    rowk = rot + (kc * 16)
                    for ic in range(128 // 16):
                        vec = plsc.load_gather(inb, [rowk, cols[ic]])
                        plsc.store_scatter(
                            ot, [prow[ic], phalf[ic] + rowk], vec)

        def fire_write(n, p):
            pltpu.async_copy(
                ot_v[p],
                tab_hbm.at[pl.ds(tb_of(n) * 64, 64)],
                wsem[p],
            )

        def wait_write(p):
            pltpu.make_async_copy(
                tab_hbm.at[pl.ds(0, 64)], ot_v[p], wsem[p]).wait()

        fire_read(0, 0)
        fire_read(1, 1)
        wait_read(0)
        transpose(0)
        fire_read(2, 0)
        fire_write(0, 0)
        wait_read(1)
        transpose(1)
        fire_read(3, 1)
        fire_write(1, 1)

        def step(t, carry):
            for p in range(2):
                n = 2 * t + p
                wait_write(p)
                wait_read(p)
                transpose(p)
                fire_read(n + 2, p)
                fire_write(n, p)
            return carry

        lax.fori_loop(1, slots // 2 - 1, step, 0)

        n = slots - 2
        wait_write(0)
        wait_read(0)
        transpose(0)
        fire_write(n, 0)
        wait_write(1)
        wait_read(1)
        transpose(1)
        fire_write(n + 1, 1)
        wait_write(0)
        wait_write(1)

        @pl.when(wid == NW - 1)
        def _tail():
            pltpu.sync_copy(tail_hbm, tbounce)
            pltpu.sync_copy(
                tbounce, tab_hbm.at[pl.ds(n_full * 64, n_tail // 2)])

    return relayout


@functools.lru_cache(maxsize=None)
def _build(B, S, V2):
    NB = B // S
    stripe = NB // NW
    units_per_j = stripe // U
    NU = S * units_per_j
    mesh = plsc.VectorSubcoreMesh(
        core_axis_name="c", subcore_axis_name="s", num_cores=NC, num_subcores=NS
    )

    @functools.partial(
        pl.kernel,
        mesh=mesh,
        out_type=jax.ShapeDtypeStruct((S, D, NB), jnp.float32),
        scratch_types=[
            pltpu.VMEM((U,), jnp.int32),
            pltpu.VMEM((U,), jnp.int32),
            pltpu.VMEM((U,), jnp.int32),
            pltpu.VMEM((U,), jnp.int32),
            pltpu.VMEM((U, 2 * D), jnp.float32),
            pltpu.VMEM((U, 2 * D), jnp.float32),
            pltpu.VMEM((D, U), jnp.float32),
            pltpu.VMEM((D, U), jnp.float32),
            pltpu.SemaphoreType.DMA,
            pltpu.SemaphoreType.DMA,
            pltpu.SemaphoreType.DMA,
            pltpu.SemaphoreType.DMA,
            pltpu.SemaphoreType.DMA,
        ],
        compiler_params=pltpu.CompilerParams(needs_layout_passes=False),
    )
    def emb(pair_hbm, half_hbm, tab_hbm, out_hbm,
            pair0, pair1, half0, half1, buf0, buf1, bt0, bt1,
            isem, g0, g1, w0, w1):
        wid = lax.axis_index("s") * NC + lax.axis_index("c")
        ubase = wid * NU
        col_base = wid * stripe
        pair_v = (pair0, pair1)
        half_v = (half0, half1)
        buf_v = (buf0, buf1)
        bt_v = (bt0, bt1)
        gsem = (g0, g1)
        wsem = (w0, w1)
        iota16 = lax.iota(jnp.int32, 16)

        def fire_idx(n, p):
            off = (ubase + n) * U
            pltpu.async_copy(pair_hbm.at[pl.ds(off, U)], pair_v[p], isem)
            pltpu.async_copy(half_hbm.at[pl.ds(off, U)], half_v[p], isem)

        def wait_idx(p):
            pltpu.make_async_copy(
                pair_hbm.at[pl.ds(0, U)], pair_v[p], isem).wait()
            pltpu.make_async_copy(
                half_hbm.at[pl.ds(0, U)], half_v[p], isem).wait()

        def fire_gathers(p):
            for c in range(U // 128):
                pltpu.async_copy(
                    tab_hbm.at[pair_v[p].at[pl.ds(c * 128, 128)]],
                    buf_v[p].at[pl.ds(c * 128, 128)],
                    gsem[p],
                )

        def wait_gathers(p):
            pltpu.make_async_copy(
                tab_hbm.at[pl.ds(0, U)], buf_v[p], gsem[p]).wait()

        def transpose(p):
            buf = buf_v[p]
            bt = bt_v[p]
            halfs = [half_v[p][pl.ds(mb * 16, 16)] for mb in range(U // 16)]
            rows = [iota16 + (mb * 16) for mb in range(U // 16)]

            @plsc.parallel_loop(0, 16)
            def body(c):
                rot = (iota16 + c) & 15
                for kb in range(D // 16):
                    rowk = rot + (kb * 16)
                    for mb in range(U // 16):
                        vec = plsc.load_gather(buf, [rows[mb], halfs[mb] + rowk])
                        plsc.store_scatter(bt, [rowk, rows[mb]], vec)

        def fire_writes(n, p):
            j = n // units_per_j
            h = n % units_per_j
            col0 = col_base + h * U
            bt = bt_v[p]
            for kb in range(D // 8):
                pltpu.async_copy(
                    bt.at[pl.ds(kb * 8, 8)],
                    out_hbm.at[j, pl.ds(kb * 8, 8), pl.ds(col0, U)],
                    wsem[p],
                )

        def wait_writes(p):
            pltpu.make_async_copy(
                out_hbm.at[0, :, pl.ds(0, U)], bt_v[p], wsem[p]).wait()

        fire_idx(0, 0)
        wait_idx(0)
        fire_gathers(0)
        fire_idx(1, 1)

        wait_gathers(0)
        wait_idx(1)
        fire_gathers(1)
        transpose(0)
        fire_idx(2, 0)
        fire_writes(0, 0)
        wait_gathers(1)
        wait_idx(0)
        fire_gathers(0)
        transpose(1)
        fire_idx(3, 1)
        fire_writes(1, 1)

        def step(t, carry):
            for p in range(2):
                n = 2 * t + p
                wait_writes(p)
                wait_gathers(p)
                wait_idx(1 - p)
                fire_gathers(1 - p)
                transpose(p)
                fire_idx(n + 2, p)
                fire_writes(n, p)
            return carry

        lax.fori_loop(1, NU // 2 - 1, step, 0)

        n = NU - 2
        wait_writes(0)
        wait_gathers(0)
        wait_idx(1)
        fire_gathers(1)
        transpose(0)
        fire_writes(n, 0)
        wait_writes(1)
        wait_gathers(1)
        transpose(1)
        fire_writes(n + 1, 1)
        wait_writes(0)
        wait_writes(1)

    return emb


def kernel(token_ids, weight):
    NB, S = token_ids.shape
    V, Dm = weight.shape
    B = NB * S
    stripe = NB // NW
    ids = token_ids.astype(jnp.int32)
    perm = ids.T.reshape(S, NW, stripe // U, U).transpose(1, 0, 2, 3).reshape(-1)
    pair = perm >> 1
    half = (perm & 1) << 6
    n_full = (V // 128) * 128
    tail = weight[n_full:].reshape((V - n_full) // 2, 2 * Dm)
    tab2 = _build_relayout(V, Dm)(weight.T, tail)
    emb = _build(B, S, V // 2)
    out = emb(pair, half, tab2)
    return out.transpose(2, 0, 1)


# speedup vs baseline: 3.1838x; 1.0846x over previous
import functools

import jax
import jax.numpy as jnp
from jax import lax
from jax.experimental import pallas as pl
from jax.experimental.pallas import tpu as pltpu
from jax.experimental.pallas import tpu_sc as plsc

NC = 2
NS = 16
NW = NC * NS

U = 256
D = 64


@functools.lru_cache(maxsize=None)
def _build_relayout(V, Dm):
    n_full = V // 128
    n_tail = V - n_full * 128
    slots = 246
    mesh = plsc.VectorSubcoreMesh(
        core_axis_name="c", subcore_axis_name="s", num_cores=NC, num_subcores=NS
    )

    @functools.partial(
        pl.kernel,
        mesh=mesh,
        out_type=jax.ShapeDtypeStruct((V // 2, 2 * Dm), jnp.float32),
        scratch_types=[
            pltpu.VMEM((Dm, 128), jnp.float32),
            pltpu.VMEM((Dm, 128), jnp.float32),
            pltpu.VMEM((64, 2 * Dm), jnp.float32),
            pltpu.VMEM((64, 2 * Dm), jnp.float32),
            pltpu.VMEM((n_tail // 2, 2 * Dm), jnp.float32),
            pltpu.SemaphoreType.DMA,
            pltpu.SemaphoreType.DMA,
            pltpu.SemaphoreType.DMA,
            pltpu.SemaphoreType.DMA,
        ],
        compiler_params=pltpu.CompilerParams(needs_layout_passes=False),
    )
    def relayout(wt_hbm, tail_hbm, tab_hbm, in0, in1, ot0, ot1, tbounce,
                 r0, r1, w0, w1):
        wid = lax.axis_index("s") * NC + lax.axis_index("c")
        in_v = (in0, in1)
        ot_v = (ot0, ot1)
        rsem = (r0, r1)
        wsem = (w0, w1)
        iota16 = lax.iota(jnp.int32, 16)

        def tb_of(n):
            raw = n * NW + wid
            return jnp.where(raw < n_full, raw, 0)

        def fire_read(n, p):
            tb = tb_of(n)
            for kb in range(Dm // 8):
                pltpu.async_copy(
                    wt_hbm.at[pl.ds(kb * 8, 8), pl.ds(tb * 128, 128)],
                    in_v[p].at[pl.ds(kb * 8, 8)],
                    rsem[p],
                )

        def wait_read(p):
            pltpu.make_async_copy(
                wt_hbm.at[pl.ds(0, Dm), pl.ds(0, 128)], in_v[p], rsem[p]).wait()

        cols = [iota16 + (ic * 16) for ic in range(128 // 16)]
        prow = [c >> 1 for c in cols]
        phalf = [(c & 1) << 6 for c in cols]

        def transpose(p):
            inb = in_v[p]
            ot = ot_v[p]

            @plsc.parallel_loop(0, 16, unroll=2)
            def body(c):
                rot = (iota16 + c) & 15
                for kc in range(Dm // 16):
                    rowk = rot + (kc * 16)
                    for ic in range(128 // 16):
                        vec = plsc.load_gather(inb, [rowk, cols[ic]])
                        plsc.store_scatter(
                            ot, [prow[ic], phalf[ic] + rowk], vec)

        def fire_write(n, p):
            pltpu.async_copy(
                ot_v[p],
                tab_hbm.at[pl.ds(tb_of(n) * 64, 64)],
                wsem[p],
            )

        def wait_write(p):
            pltpu.make_async_copy(
                tab_hbm.at[pl.ds(0, 64)], ot_v[p], wsem[p]).wait()

        fire_read(0, 0)
        fire_read(1, 1)
        wait_read(0)
        transpose(0)
        fire_read(2, 0)
        fire_write(0, 0)
        wait_read(1)
        transpose(1)
        fire_read(3, 1)
        fire_write(1, 1)

        def step(t, carry):
            for p in range(2):
                n = 2 * t + p
                wait_write(p)
                wait_read(p)
                transpose(p)
                fire_read(n + 2, p)
                fire_write(n, p)
            return carry

        lax.fori_loop(1, slots // 2 - 1, step, 0)

        n = slots - 2
        wait_write(0)
        wait_read(0)
        transpose(0)
        fire_write(n, 0)
        wait_write(1)
        wait_read(1)
        transpose(1)
        fire_write(n + 1, 1)
        wait_write(0)
        wait_write(1)

        @pl.when(wid == NW - 1)
        def _tail():
            pltpu.sync_copy(tail_hbm, tbounce)
            pltpu.sync_copy(
                tbounce, tab_hbm.at[pl.ds(n_full * 64, n_tail // 2)])

    return relayout


@functools.lru_cache(maxsize=None)
def _build(B, S, V2):
    NB = B // S
    stripe = NB // NW
    units_per_j = stripe // U
    NU = S * units_per_j
    mesh = plsc.VectorSubcoreMesh(
        core_axis_name="c", subcore_axis_name="s", num_cores=NC, num_subcores=NS
    )

    @functools.partial(
        pl.kernel,
        mesh=mesh,
        out_type=jax.ShapeDtypeStruct((S, D, NB), jnp.float32),
        scratch_types=[
            pltpu.VMEM((U,), jnp.int32),
            pltpu.VMEM((U,), jnp.int32),
            pltpu.VMEM((U,), jnp.int32),
            pltpu.VMEM((U,), jnp.int32),
            pltpu.VMEM((U, 2 * D), jnp.float32),
            pltpu.VMEM((U, 2 * D), jnp.float32),
            pltpu.VMEM((D, U), jnp.float32),
            pltpu.VMEM((D, U), jnp.float32),
            pltpu.SemaphoreType.DMA,
            pltpu.SemaphoreType.DMA,
            pltpu.SemaphoreType.DMA,
            pltpu.SemaphoreType.DMA,
            pltpu.SemaphoreType.DMA,
        ],
        compiler_params=pltpu.CompilerParams(needs_layout_passes=False),
    )
    def emb(pair_hbm, half_hbm, tab_hbm, out_hbm,
            pair0, pair1, half0, half1, buf0, buf1, bt0, bt1,
            isem, g0, g1, w0, w1):
        wid = lax.axis_index("s") * NC + lax.axis_index("c")
        ubase = wid * NU
        col_base = wid * stripe
        pair_v = (pair0, pair1)
        half_v = (half0, half1)
        buf_v = (buf0, buf1)
        bt_v = (bt0, bt1)
        gsem = (g0, g1)
        wsem = (w0, w1)
        iota16 = lax.iota(jnp.int32, 16)

        def fire_idx(n, p):
            off = (ubase + n) * U
            pltpu.async_copy(pair_hbm.at[pl.ds(off, U)], pair_v[p], isem)
            pltpu.async_copy(half_hbm.at[pl.ds(off, U)], half_v[p], isem)

        def wait_idx(p):
            pltpu.make_async_copy(
                pair_hbm.at[pl.ds(0, U)], pair_v[p], isem).wait()
            pltpu.make_async_copy(
                half_hbm.at[pl.ds(0, U)], half_v[p], isem).wait()

        def fire_gathers(p):
            for c in range(U // 128):
                pltpu.async_copy(
                    tab_hbm.at[pair_v[p].at[pl.ds(c * 128, 128)]],
                    buf_v[p].at[pl.ds(c * 128, 128)],
                    gsem[p],
                )

        def wait_gathers(p):
            pltpu.make_async_copy(
                tab_hbm.at[pl.ds(0, U)], buf_v[p], gsem[p]).wait()

        def transpose(p):
            buf = buf_v[p]
            bt = bt_v[p]
            halfs = [half_v[p][pl.ds(mb * 16, 16)] for mb in range(U // 16)]
            rows = [iota16 + (mb * 16) for mb in range(U // 16)]

            @plsc.parallel_loop(0, 16, unroll=2)
            def body(c):
                rot = (iota16 + c) & 15
                for kb in range(D // 16):
                    rowk = rot + (kb * 16)
                    for mb in range(U // 16):
                        vec = plsc.load_gather(buf, [rows[mb], halfs[mb] + rowk])
                        plsc.store_scatter(bt, [rowk, rows[mb]], vec)

        def fire_writes(n, p):
            j = n // units_per_j
            h = n % units_per_j
            col0 = col_base + h * U
            bt = bt_v[p]
            for kb in range(D // 8):
                pltpu.async_copy(
                    bt.at[pl.ds(kb * 8, 8)],
                    out_hbm.at[j, pl.ds(kb * 8, 8), pl.ds(col0, U)],
                    wsem[p],
                )

        def wait_writes(p):
            pltpu.make_async_copy(
                out_hbm.at[0, :, pl.ds(0, U)], bt_v[p], wsem[p]).wait()

        fire_idx(0, 0)
        wait_idx(0)
        fire_gathers(0)
        fire_idx(1, 1)

        wait_gathers(0)
        wait_idx(1)
        fire_gathers(1)
        transpose(0)
        fire_idx(2, 0)
        fire_writes(0, 0)
        wait_gathers(1)
        wait_idx(0)
        fire_gathers(0)
        transpose(1)
        fire_idx(3, 1)
        fire_writes(1, 1)

        def step(t, carry):
            for p in range(2):
                n = 2 * t + p
                wait_writes(p)
                wait_gathers(p)
                wait_idx(1 - p)
                fire_gathers(1 - p)
                transpose(p)
                fire_idx(n + 2, p)
                fire_writes(n, p)
            return carry

        lax.fori_loop(1, NU // 2 - 1, step, 0)

        n = NU - 2
        wait_writes(0)
        wait_gathers(0)
        wait_idx(1)
        fire_gathers(1)
        transpose(0)
        fire_writes(n, 0)
        wait_writes(1)
        wait_gathers(1)
        transpose(1)
        fire_writes(n + 1, 1)
        wait_writes(0)
        wait_writes(1)

    return emb


def kernel(token_ids, weight):
    NB, S = token_ids.shape
    V, Dm = weight.shape
    B = NB * S
    stripe = NB // NW
    ids = token_ids.astype(jnp.int32)
    perm = ids.T.reshape(S, NW, stripe // U, U).transpose(1, 0, 2, 3).reshape(-1)
    pair = perm >> 1
    half = (perm & 1) << 6
    n_full = (V // 128) * 128
    tail = weight[n_full:].reshape((V - n_full) // 2, 2 * Dm)
    tab2 = _build_relayout(V, Dm)(weight.T, tail)
    emb = _build(B, S, V // 2)
    out = emb(pair, half, tab2)
    return out.transpose(2, 0, 1)


# speedup vs baseline: 3.6572x; 1.1487x over previous
import functools

import jax
import jax.numpy as jnp
from jax import lax
from jax.experimental import pallas as pl
from jax.experimental.pallas import tpu as pltpu
from jax.experimental.pallas import tpu_sc as plsc

NC = 2
NS = 16
NW = NC * NS

U = 256
D = 64


@functools.lru_cache(maxsize=None)
def _build_relayout(V, Dm):
    n_full = V // 128
    n_tail = V - n_full * 128
    slots = 246
    mesh = plsc.VectorSubcoreMesh(
        core_axis_name="c", subcore_axis_name="s", num_cores=NC, num_subcores=NS
    )

    @functools.partial(
        pl.kernel,
        mesh=mesh,
        out_type=jax.ShapeDtypeStruct((V // 2, 2 * Dm), jnp.float32),
        scratch_types=[
            pltpu.VMEM((Dm, 128), jnp.float32),
            pltpu.VMEM((Dm, 128), jnp.float32),
            pltpu.VMEM((64, 2 * Dm), jnp.float32),
            pltpu.VMEM((64, 2 * Dm), jnp.float32),
            pltpu.VMEM((n_tail // 2, 2 * Dm), jnp.float32),
            pltpu.SemaphoreType.DMA,
            pltpu.SemaphoreType.DMA,
            pltpu.SemaphoreType.DMA,
            pltpu.SemaphoreType.DMA,
        ],
        compiler_params=pltpu.CompilerParams(needs_layout_passes=False),
    )
    def relayout(wt_hbm, tail_hbm, tab_hbm, in0, in1, ot0, ot1, tbounce,
                 r0, r1, w0, w1):
        wid = lax.axis_index("s") * NC + lax.axis_index("c")
        in_v = (in0, in1)
        ot_v = (ot0, ot1)
        rsem = (r0, r1)
        wsem = (w0, w1)
        iota16 = lax.iota(jnp.int32, 16)

        def tb_of(n):
            raw = n * NW + wid
            return jnp.where(raw < n_full, raw, 0)

        def fire_read(n, p):
            tb = tb_of(n)
            for kb in range(Dm // 8):
                pltpu.async_copy(
                    wt_hbm.at[pl.ds(kb * 8, 8), pl.ds(tb * 128, 128)],
                    in_v[p].at[pl.ds(kb * 8, 8)],
                    rsem[p],
                )

        def wait_read(p):
            pltpu.make_async_copy(
                wt_hbm.at[pl.ds(0, Dm), pl.ds(0, 128)], in_v[p], rsem[p]).wait()

        cols = [iota16 + (ic * 16) for ic in range(128 // 16)]
        prow = [c >> 1 for c in cols]
        phalf = [(c & 1) << 6 for c in cols]

        def transpose(p):
            inb = in_v[p]
            ot = ot_v[p]

            @plsc.parallel_loop(0, 16, unroll=4)
            def body(c):
                rot = (iota16 + c) & 15
                for kc in range(Dm // 16):
                    rowk = rot + (kc * 16)
                    for ic in range(128 // 16):
                        vec = plsc.load_gather(inb, [rowk, cols[ic]])
                        plsc.store_scatter(
                            ot, [prow[ic], phalf[ic] + rowk], vec)

        def fire_write(n, p):
            pltpu.async_copy(
                ot_v[p],
                tab_hbm.at[pl.ds(tb_of(n) * 64, 64)],
                wsem[p],
            )

        def wait_write(p):
            pltpu.make_async_copy(
                tab_hbm.at[pl.ds(0, 64)], ot_v[p], wsem[p]).wait()

        fire_read(0, 0)
        fire_read(1, 1)
        wait_read(0)
        transpose(0)
        fire_read(2, 0)
        fire_write(0, 0)
        wait_read(1)
        transpose(1)
        fire_read(3, 1)
        fire_write(1, 1)

        def step(t, carry):
            for p in range(2):
                n = 2 * t + p
                wait_write(p)
                wait_read(p)
                transpose(p)
                fire_read(n + 2, p)
                fire_write(n, p)
            return carry

        lax.fori_loop(1, slots // 2 - 1, step, 0)

        n = slots - 2
        wait_write(0)
        wait_read(0)
        transpose(0)
        fire_write(n, 0)
        wait_write(1)
        wait_read(1)
        transpose(1)
        fire_write(n + 1, 1)
        wait_write(0)
        wait_write(1)

        @pl.when(wid == NW - 1)
        def _tail():
            pltpu.sync_copy(tail_hbm, tbounce)
            pltpu.sync_copy(
                tbounce, tab_hbm.at[pl.ds(n_full * 64, n_tail // 2)])

    return relayout


@functools.lru_cache(maxsize=None)
def _build(B, S, V2):
    NB = B // S
    stripe = NB // NW
    units_per_j = stripe // U
    NU = S * units_per_j
    mesh = plsc.VectorSubcoreMesh(
        core_axis_name="c", subcore_axis_name="s", num_cores=NC, num_subcores=NS
    )

    @functools.partial(
        pl.kernel,
        mesh=mesh,
        out_type=jax.ShapeDtypeStruct((S, D, NB), jnp.float32),
        scratch_types=[
            pltpu.VMEM((U,), jnp.int32),
            pltpu.VMEM((U,), jnp.int32),
            pltpu.VMEM((U,), jnp.int32),
            pltpu.VMEM((U,), jnp.int32),
            pltpu.VMEM((U, 2 * D), jnp.float32),
            pltpu.VMEM((U, 2 * D), jnp.float32),
            pltpu.VMEM((D, U), jnp.float32),
            pltpu.VMEM((D, U), jnp.float32),
            pltpu.SemaphoreType.DMA,
            pltpu.SemaphoreType.DMA,
            pltpu.SemaphoreType.DMA,
            pltpu.SemaphoreType.DMA,
            pltpu.SemaphoreType.DMA,
        ],
        compiler_params=pltpu.CompilerParams(needs_layout_passes=False),
    )
    def emb(pair_hbm, half_hbm, tab_hbm, out_hbm,
            pair0, pair1, half0, half1, buf0, buf1, bt0, bt1,
            isem, g0, g1, w0, w1):
        wid = lax.axis_index("s") * NC + lax.axis_index("c")
        ubase = wid * NU
        col_base = wid * stripe
        pair_v = (pair0, pair1)
        half_v = (half0, half1)
        buf_v = (buf0, buf1)
        bt_v = (bt0, bt1)
        gsem = (g0, g1)
        wsem = (w0, w1)
        iota16 = lax.iota(jnp.int32, 16)

        def fire_idx(n, p):
            off = (ubase + n) * U
            pltpu.async_copy(pair_hbm.at[pl.ds(off, U)], pair_v[p], isem)
            pltpu.async_copy(half_hbm.at[pl.ds(off, U)], half_v[p], isem)

        def wait_idx(p):
            pltpu.make_async_copy(
                pair_hbm.at[pl.ds(0, U)], pair_v[p], isem).wait()
            pltpu.make_async_copy(
                half_hbm.at[pl.ds(0, U)], half_v[p], isem).wait()

        def fire_gathers(p):
            for c in range(U // 128):
                pltpu.async_copy(
                    tab_hbm.at[pair_v[p].at[pl.ds(c * 128, 128)]],
                    buf_v[p].at[pl.ds(c * 128, 128)],
                    gsem[p],
                )

        def wait_gathers(p):
            pltpu.make_async_copy(
                tab_hbm.at[pl.ds(0, U)], buf_v[p], gsem[p]).wait()

        def transpose(p):
            buf = buf_v[p]
            bt = bt_v[p]
            halfs = [half_v[p][pl.ds(mb * 16, 16)] for mb in range(U // 16)]
            rows = [iota16 + (mb * 16) for mb in range(U // 16)]

            @plsc.parallel_loop(0, 16, unroll=4)
            def body(c):
                rot = (iota16 + c) & 15
                for kb in range(D // 16):
                    rowk = rot + (kb * 16)
                    for mb in range(U // 16):
                        vec = plsc.load_gather(buf, [rows[mb], halfs[mb] + rowk])
                        plsc.store_scatter(bt, [rowk, rows[mb]], vec)

        def fire_writes(n, p):
            j = n // units_per_j
            h = n % units_per_j
            col0 = col_base + h * U
            bt = bt_v[p]
            for kb in range(D // 8):
                pltpu.async_copy(
                    bt.at[pl.ds(kb * 8, 8)],
                    out_hbm.at[j, pl.ds(kb * 8, 8), pl.ds(col0, U)],
                    wsem[p],
                )

        def wait_writes(p):
            pltpu.make_async_copy(
                out_hbm.at[0, :, pl.ds(0, U)], bt_v[p], wsem[p]).wait()

        fire_idx(0, 0)
        wait_idx(0)
        fire_gathers(0)
        fire_idx(1, 1)

        wait_gathers(0)
        wait_idx(1)
        fire_gathers(1)
        transpose(0)
        fire_idx(2, 0)
        fire_writes(0, 0)
        wait_gathers(1)
        wait_idx(0)
        fire_gathers(0)
        transpose(1)
        fire_idx(3, 1)
        fire_writes(1, 1)

        def step(t, carry):
            for p in range(2):
                n = 2 * t + p
                wait_writes(p)
                wait_gathers(p)
                wait_idx(1 - p)
                fire_gathers(1 - p)
                transpose(p)
                fire_idx(n + 2, p)
                fire_writes(n, p)
            return carry

        lax.fori_loop(1, NU // 2 - 1, step, 0)

        n = NU - 2
        wait_writes(0)
        wait_gathers(0)
        wait_idx(1)
        fire_gathers(1)
        transpose(0)
        fire_writes(n, 0)
        wait_writes(1)
        wait_gathers(1)
        transpose(1)
        fire_writes(n + 1, 1)
        wait_writes(0)
        wait_writes(1)

    return emb


def kernel(token_ids, weight):
    NB, S = token_ids.shape
    V, Dm = weight.shape
    B = NB * S
    stripe = NB // NW
    ids = token_ids.astype(jnp.int32)
    perm = ids.T.reshape(S, NW, stripe // U, U).transpose(1, 0, 2, 3).reshape(-1)
    pair = perm >> 1
    half = (perm & 1) << 6
    n_full = (V // 128) * 128
    tail = weight[n_full:].reshape((V - n_full) // 2, 2 * Dm)
    tab2 = _build_relayout(V, Dm)(weight.T, tail)
    emb = _build(B, S, V // 2)
    out = emb(pair, half, tab2)
    return out.transpose(2, 0, 1)
